# baseline (speedup 1.0000x reference)
import functools

import jax
import jax.numpy as jnp
from jax import lax
from jax.experimental import pallas as pl
from jax.experimental.pallas import tpu as pltpu
from jax.experimental.pallas import tpu_sc as plsc

VOCAB = 100000
EMBED = 64
BATCH = 1024
ROW = 128
SPLIT = 50048

NUM_SC_CORES = 2
NUM_SC_SUBCORES = 16
NUM_WORKERS = NUM_SC_CORES * NUM_SC_SUBCORES
ROWS_PER_WORKER = BATCH // NUM_WORKERS

P_TILE = 2944
V_TILE = 5632

P_STEPS = SPLIT // P_TILE


def _prep_block(low_ref, high_ref, out_ref):
    out_ref[:, 0:EMBED] = jnp.maximum(low_ref[...], 0.0).T
    out_ref[:, EMBED:ROW] = jnp.maximum(high_ref[...], 0.0).T


def _tc_prep(tab_t):
    return pl.pallas_call(
        _prep_block,
        grid=(P_STEPS,),
        in_specs=[
            pl.BlockSpec((EMBED, P_TILE), lambda j: (0, j)),
            pl.BlockSpec((EMBED, P_TILE), lambda j: (0, j + P_STEPS)),
        ],
        out_specs=pl.BlockSpec((P_TILE, ROW), lambda j: (j, 0)),
        out_shape=jax.ShapeDtypeStruct((SPLIT, ROW), jnp.float32),
    )(tab_t, tab_t)


def _sc_gather(tab128, idx):
    mesh = plsc.VectorSubcoreMesh(core_axis_name="c", subcore_axis_name="s")

    @functools.partial(
        pl.kernel,
        mesh=mesh,
        out_type=jax.ShapeDtypeStruct((BATCH, ROW), jnp.float32),
        scratch_types=[
            pltpu.VMEM((ROWS_PER_WORKER,), jnp.int32),
            pltpu.VMEM((ROWS_PER_WORKER, ROW), jnp.float32),
            pltpu.SemaphoreType.DMA,
        ],
    )
    def gather_kernel(tab_hbm, idx_hbm, out_hbm, idx_v, rows_v, sem):
        wid = lax.axis_index("s") * NUM_SC_CORES + lax.axis_index("c")
        base = wid * ROWS_PER_WORKER
        pltpu.sync_copy(idx_hbm.at[pl.ds(base, ROWS_PER_WORKER)], idx_v)
        pltpu.async_copy(tab_hbm.at[idx_v], rows_v, sem).wait()
        pltpu.sync_copy(rows_v, out_hbm.at[pl.ds(base, ROWS_PER_WORKER)])

    return gather_kernel(tab128, idx)


def _fc_block_t(act2_ref, sel_ref, w_ref, b_ref, out_ref):
    a2 = act2_ref[...]
    sel = sel_ref[...]
    low = a2[:, :EMBED]
    high = a2[:, EMBED:]
    a = jnp.where(sel != 0.0, high, low).astype(jnp.bfloat16)
    w = w_ref[...].astype(jnp.bfloat16)
    ones = jnp.ones((1, BATCH), jnp.float32)
    out_ref[...] = lax.dot_general(
        w, a, (((0,), (1,)), ((), ())),
        preferred_element_type=jnp.float32,
    ) + lax.dot_general(
        b_ref[...], ones, (((0,), (0,)), ((), ())),
        preferred_element_type=jnp.float32,
    )


def _tc_project_t(act2, sel, w_t, fc_bias2d):
    grid = (pl.cdiv(VOCAB, V_TILE),)
    return pl.pallas_call(
        _fc_block_t,
        grid=grid,
        in_specs=[
            pl.BlockSpec((BATCH, ROW), lambda j: (0, 0)),
            pl.BlockSpec((BATCH, 1), lambda j: (0, 0)),
            pl.BlockSpec((EMBED, V_TILE), lambda j: (0, j)),
            pl.BlockSpec((1, V_TILE), lambda j: (0, j)),
        ],
        out_specs=pl.BlockSpec((V_TILE, BATCH), lambda j: (j, 0)),
        out_shape=jax.ShapeDtypeStruct((VOCAB, BATCH), jnp.float32),
    )(act2, sel, w_t, fc_bias2d)


def kernel(text, emb_table, fc_weight, fc_bias):
    text = text.astype(jnp.int32)
    bias2d = fc_bias.reshape(1, VOCAB)
    is_high = text >= SPLIT
    gidx = jnp.where(is_high, text - SPLIT, text)
    sel = is_high.astype(jnp.float32).reshape(BATCH, 1)
    tab_t = emb_table.T
    w_t = fc_weight.T
    tab128 = _tc_prep(tab_t)
    act2 = _sc_gather(tab128, gidx)
    out_t = _tc_project_t(act2, sel, w_t, bias2d)
    return out_t.T


# speedup vs baseline: 1.0215x; 1.0215x over previous
import functools

import jax
import jax.numpy as jnp
from jax import lax
from jax.experimental import pallas as pl
from jax.experimental.pallas import tpu as pltpu
from jax.experimental.pallas import tpu_sc as plsc

VOCAB = 100000
EMBED = 64
BATCH = 1024
ROW = 128
SPLIT = 50176

NUM_SC_CORES = 2
NUM_SC_SUBCORES = 16
NUM_WORKERS = NUM_SC_CORES * NUM_SC_SUBCORES
ROWS_PER_WORKER = BATCH // NUM_WORKERS

P_TILE = 6272
V_TILE = 5632

P_STEPS = SPLIT // P_TILE


def _prep_block(low_ref, high_ref, out_ref):
    out_ref[:, 0:EMBED] = jnp.maximum(low_ref[...], 0.0).T
    out_ref[:, EMBED:ROW] = jnp.maximum(high_ref[...], 0.0).T


def _tc_prep(tab_t):
    return pl.pallas_call(
        _prep_block,
        grid=(P_STEPS,),
        in_specs=[
            pl.BlockSpec((EMBED, P_TILE), lambda j: (0, j)),
            pl.BlockSpec((EMBED, P_TILE), lambda j: (0, j + P_STEPS)),
        ],
        out_specs=pl.BlockSpec((P_TILE, ROW), lambda j: (j, 0)),
        out_shape=jax.ShapeDtypeStruct((SPLIT, ROW), jnp.float32),
    )(tab_t, tab_t)


def _sc_gather(tab128, idx):
    mesh = plsc.VectorSubcoreMesh(core_axis_name="c", subcore_axis_name="s")

    @functools.partial(
        pl.kernel,
        mesh=mesh,
        out_type=jax.ShapeDtypeStruct((BATCH, ROW), jnp.float32),
        scratch_types=[
            pltpu.VMEM((ROWS_PER_WORKER,), jnp.int32),
            pltpu.VMEM((ROWS_PER_WORKER, ROW), jnp.float32),
            pltpu.SemaphoreType.DMA,
        ],
    )
    def gather_kernel(tab_hbm, idx_hbm, out_hbm, idx_v, rows_v, sem):
        wid = lax.axis_index("s") * NUM_SC_CORES + lax.axis_index("c")
        base = wid * ROWS_PER_WORKER
        pltpu.sync_copy(idx_hbm.at[pl.ds(base, ROWS_PER_WORKER)], idx_v)
        pltpu.async_copy(tab_hbm.at[idx_v], rows_v, sem).wait()
        pltpu.sync_copy(rows_v, out_hbm.at[pl.ds(base, ROWS_PER_WORKER)])

    return gather_kernel(tab128, idx)


def _fc_block_t(act2_ref, sel_ref, w_ref, b_ref, out_ref):
    a2 = act2_ref[...]
    sel = sel_ref[...]
    low = a2[:, :EMBED]
    high = a2[:, EMBED:]
    a = jnp.where(sel != 0.0, high, low).astype(jnp.bfloat16)
    w = w_ref[...].astype(jnp.bfloat16)
    ones = jnp.ones((1, BATCH), jnp.float32)
    out_ref[...] = lax.dot_general(
        w, a, (((0,), (1,)), ((), ())),
        preferred_element_type=jnp.float32,
    ) + lax.dot_general(
        b_ref[...], ones, (((0,), (0,)), ((), ())),
        preferred_element_type=jnp.float32,
    )


def _tc_project_t(act2, sel, w_t, fc_bias2d):
    grid = (pl.cdiv(VOCAB, V_TILE),)
    return pl.pallas_call(
        _fc_block_t,
        grid=grid,
        in_specs=[
            pl.BlockSpec((BATCH, ROW), lambda j: (0, 0)),
            pl.BlockSpec((BATCH, 1), lambda j: (0, 0)),
            pl.BlockSpec((EMBED, V_TILE), lambda j: (0, j)),
            pl.BlockSpec((1, V_TILE), lambda j: (0, j)),
        ],
        out_specs=pl.BlockSpec((V_TILE, BATCH), lambda j: (j, 0)),
        out_shape=jax.ShapeDtypeStruct((VOCAB, BATCH), jnp.float32),
    )(act2, sel, w_t, fc_bias2d)


def kernel(text, emb_table, fc_weight, fc_bias):
    text = text.astype(jnp.int32)
    bias2d = fc_bias.reshape(1, VOCAB)
    is_high = text >= SPLIT
    gidx = jnp.where(is_high, text - SPLIT, text)
    sel = is_high.astype(jnp.float32).reshape(BATCH, 1)
    tab_t = emb_table.T
    w_t = fc_weight.T
    tab128 = _tc_prep(tab_t)
    act2 = _sc_gather(tab128, gidx)
    out_t = _tc_project_t(act2, sel, w_t, bias2d)
    return out_t.T


# speedup vs baseline: 1.0290x; 1.0074x over previous
import functools

import jax
import jax.numpy as jnp
from jax import lax
from jax.experimental import pallas as pl
from jax.experimental.pallas import tpu as pltpu
from jax.experimental.pallas import tpu_sc as plsc

VOCAB = 100000
EMBED = 64
BATCH = 1024
ROW = 128

NUM_SC_CORES = 2
NUM_SC_SUBCORES = 16
NUM_WORKERS = NUM_SC_CORES * NUM_SC_SUBCORES
ROWS_PER_WORKER = BATCH // NUM_WORKERS

P_TILE = 16384
V_TILE = 5632


def _prep_block(tab_t_ref, out_ref):
    out_ref[:, 0:EMBED] = jnp.maximum(tab_t_ref[...], 0.0).T


def _tc_prep(tab_t):
    grid = (pl.cdiv(VOCAB, P_TILE),)
    return pl.pallas_call(
        _prep_block,
        grid=grid,
        in_specs=[pl.BlockSpec((EMBED, P_TILE), lambda j: (0, j))],
        out_specs=pl.BlockSpec((P_TILE, ROW), lambda j: (j, 0)),
        out_shape=jax.ShapeDtypeStruct((VOCAB, ROW), jnp.float32),
    )(tab_t)


def _sc_gather(tab128, idx):
    mesh = plsc.VectorSubcoreMesh(core_axis_name="c", subcore_axis_name="s")

    @functools.partial(
        pl.kernel,
        mesh=mesh,
        out_type=jax.ShapeDtypeStruct((BATCH, ROW), jnp.float32),
        scratch_types=[
            pltpu.VMEM((ROWS_PER_WORKER,), jnp.int32),
            pltpu.VMEM((ROWS_PER_WORKER, ROW), jnp.float32),
            pltpu.SemaphoreType.DMA,
        ],
    )
    def gather_kernel(tab_hbm, idx_hbm, out_hbm, idx_v, rows_v, sem):
        wid = lax.axis_index("s") * NUM_SC_CORES + lax.axis_index("c")
        base = wid * ROWS_PER_WORKER
        pltpu.sync_copy(idx_hbm.at[pl.ds(base, ROWS_PER_WORKER)], idx_v)
        pltpu.async_copy(tab_hbm.at[idx_v], rows_v, sem).wait()
        pltpu.sync_copy(rows_v, out_hbm.at[pl.ds(base, ROWS_PER_WORKER)])

    return gather_kernel(tab128, idx)


def _fc_block_t(act2_ref, w_ref, b_ref, out_ref):
    a = act2_ref[:, 0:EMBED].astype(jnp.bfloat16)
    w = w_ref[...].astype(jnp.bfloat16)
    ones = jnp.ones((1, BATCH), jnp.float32)
    out_ref[...] = lax.dot_general(
        w, a, (((0,), (1,)), ((), ())),
        preferred_element_type=jnp.float32,
    ) + lax.dot_general(
        b_ref[...], ones, (((0,), (0,)), ((), ())),
        preferred_element_type=jnp.float32,
    )


def _tc_project_t(act2, w_t, fc_bias2d):
    grid = (pl.cdiv(VOCAB, V_TILE),)
    return pl.pallas_call(
        _fc_block_t,
        grid=grid,
        in_specs=[
            pl.BlockSpec((BATCH, ROW), lambda j: (0, 0)),
            pl.BlockSpec((EMBED, V_TILE), lambda j: (0, j)),
            pl.BlockSpec((1, V_TILE), lambda j: (0, j)),
        ],
        out_specs=pl.BlockSpec((V_TILE, BATCH), lambda j: (j, 0)),
        out_shape=jax.ShapeDtypeStruct((VOCAB, BATCH), jnp.float32),
    )(act2, w_t, fc_bias2d)


def kernel(text, emb_table, fc_weight, fc_bias):
    text = text.astype(jnp.int32)
    tab_t = emb_table.T
    w_t = fc_weight.T
    tab128 = _tc_prep(tab_t)
    act2 = _sc_gather(tab128, text)
    out_t = _tc_project_t(act2, w_t, fc_bias.reshape(1, VOCAB))
    return out_t.T


# speedup vs baseline: 1.0409x; 1.0115x over previous
import functools

import jax
import jax.numpy as jnp
from jax import lax
from jax.experimental import pallas as pl
from jax.experimental.pallas import tpu as pltpu
from jax.experimental.pallas import tpu_sc as plsc

VOCAB = 100000
EMBED = 64
BATCH = 1024
ROW = 128

NUM_SC_CORES = 2
NUM_SC_SUBCORES = 16
NUM_WORKERS = NUM_SC_CORES * NUM_SC_SUBCORES
ROWS_PER_WORKER = BATCH // NUM_WORKERS

P_TILE = 16384
V_TILE = 5632


def _prep_block(tab_t_ref, out_ref):
    t = jnp.maximum(tab_t_ref[...], 0.0).astype(jnp.bfloat16).T
    out_ref[:, 0:EMBED] = t.astype(jnp.float32)


def _tc_prep(tab_t):
    grid = (pl.cdiv(VOCAB, P_TILE),)
    return pl.pallas_call(
        _prep_block,
        grid=grid,
        in_specs=[pl.BlockSpec((EMBED, P_TILE), lambda j: (0, j))],
        out_specs=pl.BlockSpec((P_TILE, ROW), lambda j: (j, 0)),
        out_shape=jax.ShapeDtypeStruct((VOCAB, ROW), jnp.float32),
    )(tab_t)


def _sc_gather(tab128, idx):
    mesh = plsc.VectorSubcoreMesh(core_axis_name="c", subcore_axis_name="s")

    @functools.partial(
        pl.kernel,
        mesh=mesh,
        out_type=jax.ShapeDtypeStruct((BATCH, ROW), jnp.float32),
        scratch_types=[
            pltpu.VMEM((ROWS_PER_WORKER,), jnp.int32),
            pltpu.VMEM((ROWS_PER_WORKER, ROW), jnp.float32),
            pltpu.SemaphoreType.DMA,
        ],
    )
    def gather_kernel(tab_hbm, idx_hbm, out_hbm, idx_v, rows_v, sem):
        wid = lax.axis_index("s") * NUM_SC_CORES + lax.axis_index("c")
        base = wid * ROWS_PER_WORKER
        pltpu.sync_copy(idx_hbm.at[pl.ds(base, ROWS_PER_WORKER)], idx_v)
        pltpu.async_copy(tab_hbm.at[idx_v], rows_v, sem).wait()
        pltpu.sync_copy(rows_v, out_hbm.at[pl.ds(base, ROWS_PER_WORKER)])

    return gather_kernel(tab128, idx)


def _fc_block_t(act2_ref, w_ref, b_ref, out_ref):
    a = act2_ref[:, 0:EMBED].astype(jnp.bfloat16)
    w = w_ref[...].astype(jnp.bfloat16)
    ones = jnp.ones((1, BATCH), jnp.float32)
    out_ref[...] = lax.dot_general(
        w, a, (((0,), (1,)), ((), ())),
        preferred_element_type=jnp.float32,
    ) + lax.dot_general(
        b_ref[...], ones, (((0,), (0,)), ((), ())),
        preferred_element_type=jnp.float32,
    )


def _tc_project_t(act2, w_t, fc_bias2d):
    grid = (pl.cdiv(VOCAB, V_TILE),)
    return pl.pallas_call(
        _fc_block_t,
        grid=grid,
        in_specs=[
            pl.BlockSpec((BATCH, ROW), lambda j: (0, 0)),
            pl.BlockSpec((EMBED, V_TILE), lambda j: (0, j)),
            pl.BlockSpec((1, V_TILE), lambda j: (0, j)),
        ],
        out_specs=pl.BlockSpec((V_TILE, BATCH), lambda j: (j, 0)),
        out_shape=jax.ShapeDtypeStruct((VOCAB, BATCH), jnp.float32),
    )(act2, w_t, fc_bias2d)


def kernel(text, emb_table, fc_weight, fc_bias):
    text = text.astype(jnp.int32)
    tab_t = emb_table.T
    w_t = fc_weight.T
    tab128 = _tc_prep(tab_t)
    act2 = _sc_gather(tab128, text)
    out_t = _tc_project_t(act2, w_t, fc_bias.reshape(1, VOCAB))
    return out_t.T


# speedup vs baseline: 1.0601x; 1.0184x over previous
import functools

import jax
import jax.numpy as jnp
from jax import lax
from jax.experimental import pallas as pl
from jax.experimental.pallas import tpu as pltpu
from jax.experimental.pallas import tpu_sc as plsc

VOCAB = 100000
EMBED = 64
BATCH = 1024
ROW = 128
SPLIT = 50176

NUM_SC_CORES = 2
NUM_SC_SUBCORES = 16
NUM_WORKERS = NUM_SC_CORES * NUM_SC_SUBCORES
ROWS_PER_WORKER = BATCH // NUM_WORKERS

P_TILE = 12544
V_TILE = 5632

P_STEPS = SPLIT // P_TILE


def _prep_block(low_ref, high_ref, out_ref):
    tl = jnp.maximum(low_ref[...], 0.0).astype(jnp.bfloat16).T
    th = jnp.maximum(high_ref[...], 0.0).astype(jnp.bfloat16).T
    out_ref[:, 0:EMBED] = tl.astype(jnp.float32)
    out_ref[:, EMBED:ROW] = th.astype(jnp.float32)


def _tc_prep(tab_t):
    return pl.pallas_call(
        _prep_block,
        grid=(P_STEPS,),
        in_specs=[
            pl.BlockSpec((EMBED, P_TILE), lambda j: (0, j)),
            pl.BlockSpec((EMBED, P_TILE), lambda j: (0, j + P_STEPS)),
        ],
        out_specs=pl.BlockSpec((P_TILE, ROW), lambda j: (j, 0)),
        out_shape=jax.ShapeDtypeStruct((SPLIT, ROW), jnp.float32),
    )(tab_t, tab_t)


def _sc_gather(tab128, idx):
    mesh = plsc.VectorSubcoreMesh(core_axis_name="c", subcore_axis_name="s")

    @functools.partial(
        pl.kernel,
        mesh=mesh,
        out_type=jax.ShapeDtypeStruct((BATCH, ROW), jnp.float32),
        scratch_types=[
            pltpu.VMEM((ROWS_PER_WORKER,), jnp.int32),
            pltpu.VMEM((ROWS_PER_WORKER, ROW), jnp.float32),
            pltpu.SemaphoreType.DMA,
        ],
    )
    def gather_kernel(tab_hbm, idx_hbm, out_hbm, idx_v, rows_v, sem):
        wid = lax.axis_index("s") * NUM_SC_CORES + lax.axis_index("c")
        base = wid * ROWS_PER_WORKER
        pltpu.sync_copy(idx_hbm.at[pl.ds(base, ROWS_PER_WORKER)], idx_v)
        pltpu.async_copy(tab_hbm.at[idx_v], rows_v, sem).wait()
        pltpu.sync_copy(rows_v, out_hbm.at[pl.ds(base, ROWS_PER_WORKER)])

    return gather_kernel(tab128, idx)


def _fc_block_t(act2_ref, sel_ref, w_ref, b_ref, out_ref):
    a2 = act2_ref[...]
    sel = sel_ref[...]
    a = jnp.where(sel != 0.0, a2[:, EMBED:], a2[:, :EMBED]).astype(jnp.bfloat16)
    w = w_ref[...].astype(jnp.bfloat16)
    ones = jnp.ones((1, BATCH), jnp.float32)
    out_ref[...] = lax.dot_general(
        w, a, (((0,), (1,)), ((), ())),
        preferred_element_type=jnp.float32,
    ) + lax.dot_general(
        b_ref[...], ones, (((0,), (0,)), ((), ())),
        preferred_element_type=jnp.float32,
    )


def _tc_project_t(act2, sel, w_t, fc_bias2d):
    grid = (pl.cdiv(VOCAB, V_TILE),)
    return pl.pallas_call(
        _fc_block_t,
        grid=grid,
        in_specs=[
            pl.BlockSpec((BATCH, ROW), lambda j: (0, 0)),
            pl.BlockSpec((BATCH, 1), lambda j: (0, 0)),
            pl.BlockSpec((EMBED, V_TILE), lambda j: (0, j)),
            pl.BlockSpec((1, V_TILE), lambda j: (0, j)),
        ],
        out_specs=pl.BlockSpec((V_TILE, BATCH), lambda j: (j, 0)),
        out_shape=jax.ShapeDtypeStruct((VOCAB, BATCH), jnp.float32),
    )(act2, sel, w_t, fc_bias2d)


def kernel(text, emb_table, fc_weight, fc_bias):
    text = text.astype(jnp.int32)
    tab_t = emb_table.T
    w_t = fc_weight.T
    is_high = text >= SPLIT
    gidx = jnp.where(is_high, text - SPLIT, text)
    sel = is_high.astype(jnp.float32).reshape(BATCH, 1)
    tab128 = _tc_prep(tab_t)
    act2 = _sc_gather(tab128, gidx)
    out_t = _tc_project_t(act2, sel, w_t, fc_bias.reshape(1, VOCAB))
    return out_t.T


# speedup vs baseline: 1.0615x; 1.0013x over previous
import functools

import jax
import jax.numpy as jnp
from jax import lax
from jax.experimental import pallas as pl
from jax.experimental.pallas import tpu as pltpu
from jax.experimental.pallas import tpu_sc as plsc

VOCAB = 100000
EMBED = 64
BATCH = 1024
ROW = 128
SPLIT = 50176

NUM_SC_CORES = 2
NUM_SC_SUBCORES = 16
NUM_WORKERS = NUM_SC_CORES * NUM_SC_SUBCORES
ROWS_PER_WORKER = BATCH // NUM_WORKERS

P_TILE = 12544
V_TILE = 5888

P_STEPS = SPLIT // P_TILE


def _prep_block(low_ref, high_ref, out_ref):
    tl = jnp.maximum(low_ref[...], 0.0).astype(jnp.bfloat16).T
    th = jnp.maximum(high_ref[...], 0.0).astype(jnp.bfloat16).T
    out_ref[:, 0:EMBED] = tl.astype(jnp.float32)
    out_ref[:, EMBED:ROW] = th.astype(jnp.float32)


def _tc_prep(tab_t):
    return pl.pallas_call(
        _prep_block,
        grid=(P_STEPS,),
        in_specs=[
            pl.BlockSpec((EMBED, P_TILE), lambda j: (0, j)),
            pl.BlockSpec((EMBED, P_TILE), lambda j: (0, j + P_STEPS)),
        ],
        out_specs=pl.BlockSpec((P_TILE, ROW), lambda j: (j, 0)),
        out_shape=jax.ShapeDtypeStruct((SPLIT, ROW), jnp.float32),
    )(tab_t, tab_t)


def _sc_gather(tab128, idx):
    mesh = plsc.VectorSubcoreMesh(core_axis_name="c", subcore_axis_name="s")

    @functools.partial(
        pl.kernel,
        mesh=mesh,
        out_type=jax.ShapeDtypeStruct((BATCH, ROW), jnp.float32),
        scratch_types=[
            pltpu.VMEM((ROWS_PER_WORKER,), jnp.int32),
            pltpu.VMEM((ROWS_PER_WORKER, ROW), jnp.float32),
            pltpu.SemaphoreType.DMA,
        ],
    )
    def gather_kernel(tab_hbm, idx_hbm, out_hbm, idx_v, rows_v, sem):
        wid = lax.axis_index("s") * NUM_SC_CORES + lax.axis_index("c")
        base = wid * ROWS_PER_WORKER
        pltpu.sync_copy(idx_hbm.at[pl.ds(base, ROWS_PER_WORKER)], idx_v)
        pltpu.async_copy(tab_hbm.at[idx_v], rows_v, sem).wait()
        pltpu.sync_copy(rows_v, out_hbm.at[pl.ds(base, ROWS_PER_WORKER)])

    return gather_kernel(tab128, idx)


def _fc_block_t(act2_ref, sel_ref, w_ref, b_ref, out_ref):
    a2 = act2_ref[...]
    sel = sel_ref[...]
    a = jnp.where(sel != 0.0, a2[:, EMBED:], a2[:, :EMBED]).astype(jnp.bfloat16)
    w = w_ref[...].astype(jnp.bfloat16)
    ones = jnp.ones((1, BATCH), jnp.float32)
    out_ref[...] = lax.dot_general(
        w, a, (((0,), (1,)), ((), ())),
        preferred_element_type=jnp.float32,
    ) + lax.dot_general(
        b_ref[...], ones, (((0,), (0,)), ((), ())),
        preferred_element_type=jnp.float32,
    )


def _tc_project_t(act2, sel, w_t, fc_bias2d):
    grid = (pl.cdiv(VOCAB, V_TILE),)
    return pl.pallas_call(
        _fc_block_t,
        grid=grid,
        in_specs=[
            pl.BlockSpec((BATCH, ROW), lambda j: (0, 0)),
            pl.BlockSpec((BATCH, 1), lambda j: (0, 0)),
            pl.BlockSpec((EMBED, V_TILE), lambda j: (0, j)),
            pl.BlockSpec((1, V_TILE), lambda j: (0, j)),
        ],
        out_specs=pl.BlockSpec((V_TILE, BATCH), lambda j: (j, 0)),
        out_shape=jax.ShapeDtypeStruct((VOCAB, BATCH), jnp.float32),
    )(act2, sel, w_t, fc_bias2d)


def kernel(text, emb_table, fc_weight, fc_bias):
    text = text.astype(jnp.int32)
    tab_t = emb_table.T
    w_t = fc_weight.T
    is_high = text >= SPLIT
    gidx = jnp.where(is_high, text - SPLIT, text)
    sel = is_high.astype(jnp.float32).reshape(BATCH, 1)
    tab128 = _tc_prep(tab_t)
    act2 = _sc_gather(tab128, gidx)
    out_t = _tc_project_t(act2, sel, w_t, fc_bias.reshape(1, VOCAB))
    return out_t.T
